# baseline (speedup 1.0000x reference)
import functools

import jax
import jax.numpy as jnp
from jax import lax
from jax.experimental import pallas as pl
from jax.experimental.pallas import tpu as pltpu
from jax.experimental.pallas import tpu_sc as plsc

_B, _S, _C = 16, 1024, 768
_K = 128


_SPG = 4


def _excl_cumsum_row(x):
    orig = x
    d = 1
    while d < _S:
        x = x + jnp.concatenate(
            [jnp.zeros((1, d), x.dtype), x[:, :-d]], axis=1)
        d *= 2
    return x - orig


def _select_body(logits_ref, sel_ref):
    brows, bcols = [], []
    for s in range(_SPG):
        lg = lax.transpose(logits_ref[s], (1, 0))
        m = jnp.max(lg, axis=-1, keepdims=True)
        e = jnp.exp(lg - m)
        ssum = jnp.sum(e, axis=-1, keepdims=True)
        vcol = 1.0 / ssum
        bcol = lax.bitcast_convert_type(vcol, jnp.int32)
        bcols.append(bcol)
        brows.append(lax.transpose(bcol, (1, 0)))
    keys = jnp.concatenate(brows, axis=0)

    def body(_, carry):
        lo, hi = carry
        mid = (lo + hi + 1) >> 1
        cnt = jnp.sum((keys >= mid).astype(jnp.int32), axis=1, keepdims=True)
        ge = cnt >= _K
        return jnp.where(ge, mid, lo), jnp.where(ge, hi, mid - 1)

    lo0 = jnp.zeros((_SPG, 1), jnp.int32)
    hi0 = jnp.full((_SPG, 1), 0x40000000, jnp.int32)
    t, _ = lax.fori_loop(0, 31, body, (lo0, hi0))

    iiK = lax.broadcasted_iota(jnp.int32, (_K, _K), 0)
    jjK = lax.broadcasted_iota(jnp.int32, (_K, _K), 1)
    rrS = lax.broadcasted_iota(jnp.int32, (_S, _K), 1)
    i2S = lax.broadcasted_iota(jnp.int32, (_S, _K), 0)

    for s in range(_SPG):
        bcol, brow, ts = bcols[s], brows[s], t[s:s + 1]
        gt_row = (brow > ts).astype(jnp.int32)
        eq_row = (brow == ts).astype(jnp.int32)
        cntG = jnp.sum(gt_row, axis=1, keepdims=True)
        pos_row = _excl_cumsum_row(gt_row)
        c_row = _excl_cumsum_row(eq_row)
        rank_tie = cntG + c_row
        tie_row = eq_row * (rank_tie < _K).astype(jnp.int32)

        gt_col = lax.transpose(gt_row, (1, 0))
        pos_col = lax.transpose(pos_row, (1, 0))
        rank_tie_col = lax.transpose(rank_tie, (1, 0))
        tie_col = lax.transpose(tie_row, (1, 0))

        m1 = (jnp.broadcast_to(pos_col, (_S, _K)) == rrS) & (
            jnp.broadcast_to(gt_col, (_S, _K)) > 0)
        ck = jnp.sum(jnp.where(m1, jnp.broadcast_to(bcol, (_S, _K)), 0),
                     axis=0, keepdims=True)
        cidx = jnp.sum(jnp.where(m1, i2S, 0), axis=0, keepdims=True)

        ckc = lax.transpose(ck, (1, 0))
        u2 = jnp.broadcast_to(ckc, (_K, _K)) + (jjK >= iiK).astype(jnp.int32)
        win2 = jnp.broadcast_to(ck, (_K, _K)) >= u2
        crank = jnp.sum(win2.astype(jnp.int32), axis=1, keepdims=True)

        cidxc = lax.transpose(cidx, (1, 0))
        m2 = (jnp.broadcast_to(crank, (_K, _K)) == jjK) & (iiK < cntG)
        selA = jnp.sum(jnp.where(m2, jnp.broadcast_to(cidxc, (_K, _K)), 0),
                       axis=0, keepdims=True)
        m3 = (jnp.broadcast_to(rank_tie_col, (_S, _K)) == rrS) & (
            jnp.broadcast_to(tie_col, (_S, _K)) > 0)
        selB = jnp.sum(jnp.where(m3, i2S, 0), axis=0, keepdims=True)
        sel_ref[s] = jnp.broadcast_to(selA + selB, (8, _K))


def _select(logits):
    lt = jnp.transpose(logits, (0, 2, 1))
    nc = logits.shape[-1]
    return pl.pallas_call(
        _select_body,
        grid=(_B // _SPG,),
        in_specs=[pl.BlockSpec((_SPG, nc, _S), lambda g: (g, 0, 0))],
        out_specs=pl.BlockSpec((_SPG, 8, _K), lambda g: (g, 0, 0)),
        out_shape=jax.ShapeDtypeStruct((_B, 8, _K), jnp.int32),
    )(lt)


def _gather(x, sel):
    info = plsc.get_sparse_core_info()
    nw = info.num_cores * info.num_subcores
    wps = nw // _B
    bpw = _K // wps
    mesh = plsc.VectorSubcoreMesh(core_axis_name="c", subcore_axis_name="s")

    @functools.partial(
        pl.kernel, mesh=mesh,
        out_type=jax.ShapeDtypeStruct((_B, _K, _C), jnp.float32),
        compiler_params=pltpu.CompilerParams(use_tc_tiling_on_sc=True),
        scratch_types=[
            pltpu.VMEM((bpw,), jnp.int32),
            pltpu.VMEM((bpw, _C), jnp.float32),
            pltpu.SemaphoreType.DMA,
        ],
    )
    def k(x_hbm, sel_hbm, out_hbm, idx_v, rows_v, sem):
        wid = lax.axis_index("s") * info.num_cores + lax.axis_index("c")
        b = wid // wps
        base = (wid % wps) * bpw
        pltpu.sync_copy(sel_hbm.at[b, 0, pl.ds(base, bpw)], idx_v)
        pltpu.async_copy(x_hbm.at[b].at[idx_v], rows_v, sem).wait()
        pltpu.sync_copy(rows_v, out_hbm.at[b, pl.ds(base, bpw)])

    return k(x, sel)


def kernel(x, logits):
    sel = _select(logits)
    return _gather(x, sel)
